# Initial kernel scaffold; baseline (speedup 1.0000x reference)
import jax
import jax.numpy as jnp
from jax.experimental import pallas as pl


def kernel(x, Wr, g1, g2, Wqkv, Wo, W1, W2, W3):
    raise NotImplementedError("write your pallas kernel here")


# speedup vs baseline: 1.4741x; 1.4741x over previous
import functools
import math

import jax
import jax.numpy as jnp
from jax import lax
from jax.experimental import pallas as pl
from jax.experimental.pallas import tpu as pltpu
from jax.experimental.pallas import tpu_sc as plsc

N_HEADS = 12
CAPACITY = 1024
_INT_MIN_PY = -2147483648


def _scores_copy_body(x_ref, wr_ref, out_ref, s_ref):
    xb = x_ref[...]
    out_ref[...] = xb
    Bb, TB, D = xb.shape
    s = lax.dot_general(xb.reshape(Bb * TB, D), wr_ref[...],
                        (((1,), (0,)), ((), ())),
                        precision=lax.Precision.HIGHEST,
                        preferred_element_type=jnp.float32)
    s_ref[...] = s.reshape(Bb, TB)


def _scores_and_copy(x, Wr):
    B, T, D = x.shape
    TB = 512
    grid = (T // TB,)
    out, scores = pl.pallas_call(
        _scores_copy_body,
        grid=grid,
        in_specs=[
            pl.BlockSpec((B, TB, D), lambda i: (0, i, 0)),
            pl.BlockSpec((D, 1), lambda i: (0, 0)),
        ],
        out_specs=[
            pl.BlockSpec((B, TB, D), lambda i: (0, i, 0)),
            pl.BlockSpec((B, TB), lambda i: (0, i)),
        ],
        out_shape=[
            jax.ShapeDtypeStruct((B, T, D), jnp.float32),
            jax.ShapeDtypeStruct((B, T), jnp.float32),
        ],
    )(x, Wr)
    return out, scores


def _prefix_excl(a, LT128, LTR):
    B, T = a.shape
    nr = T // 128
    a3 = a.reshape(B, nr, 128)
    p1 = lax.dot_general(a3, LT128, (((2,), (0,)), ((), ())),
                         preferred_element_type=jnp.float32)
    rowsum = jnp.sum(a3, axis=2)
    rowpref = lax.dot_general(rowsum, LTR, (((1,), (0,)), ((), ())),
                              preferred_element_type=jnp.float32)
    return (p1 + rowpref[:, :, None]).reshape(B, T)


def _topk_body(s_ref, idx_ref):
    s = s_ref[...]
    B, T = s.shape
    K = CAPACITY
    INT_MIN = jnp.int32(_INT_MIN_PY)
    bits = lax.bitcast_convert_type(s, jnp.int32)
    key = jnp.where(bits >= 0, bits, INT_MIN - bits)

    nr = T // 128
    i0 = lax.broadcasted_iota(jnp.int32, (128, 128), 0)
    i1 = lax.broadcasted_iota(jnp.int32, (128, 128), 1)
    LT128 = jnp.where(i0 < i1, 1.0, 0.0)
    r0 = lax.broadcasted_iota(jnp.int32, (nr, nr), 0)
    r1 = lax.broadcasted_iota(jnp.int32, (nr, nr), 1)
    LTR = jnp.where(r0 < r1, 1.0, 0.0)

    acc = jnp.zeros((B, 1), jnp.int32)
    for bit in range(31, -1, -1):
        cand_u = acc | (INT_MIN if bit == 31 else jnp.int32(1 << bit))
        cand_s = cand_u ^ INT_MIN
        cnt = jnp.sum(jnp.where(key >= cand_s, 1.0, 0.0), axis=1,
                      keepdims=True)
        acc = jnp.where(cnt >= K, cand_u, acc)
    vk = acc ^ INT_MIN

    gt = key > vk
    m = jnp.sum(jnp.where(gt, 1.0, 0.0), axis=1, keepdims=True)
    need = K - m
    tie = key == vk
    tie_pref = _prefix_excl(jnp.where(tie, 1.0, 0.0), LT128, LTR)
    sel = gt | (tie & (tie_pref < need))
    sel_f = jnp.where(sel, 1.0, 0.0)
    slot = _prefix_excl(sel_f, LT128, LTR)

    hi_f = lax.shift_right_arithmetic(key, 12).astype(jnp.float32)
    lo_f = (key & 0xFFF).astype(jnp.float32)
    gi_f = lax.broadcasted_iota(jnp.int32, (B, T), 1).astype(jnp.float32)

    CH = 1024
    cap_io0 = lax.broadcasted_iota(jnp.int32, (K, CH), 0).astype(jnp.float32)
    rows = []
    for b in range(B):
        acc_row = jnp.zeros((3, K), jnp.float32)
        acc_col = jnp.zeros((K, 3), jnp.float32)
        for c in range(T // CH):
            c0, c1 = c * CH, (c + 1) * CH
            sl = slot[b:b + 1, c0:c1]
            se = sel_f[b:b + 1, c0:c1]
            O = jnp.where((cap_io0 == sl) & (se > 0.5), 1.0, 0.0)
            V = jnp.concatenate([
                hi_f[b:b + 1, c0:c1],
                lo_f[b:b + 1, c0:c1],
                gi_f[b:b + 1, c0:c1],
            ], axis=0)
            acc_row = acc_row + lax.dot_general(
                V, O, (((1,), (1,)), ((), ())),
                precision=lax.Precision.HIGHEST,
                preferred_element_type=jnp.float32)
            acc_col = acc_col + lax.dot_general(
                O, V, (((1,), (1,)), ((), ())),
                precision=lax.Precision.HIGHEST,
                preferred_element_type=jnp.float32)
        k_row = (acc_row[0:1, :].astype(jnp.int32) * 4096
                 + acc_row[1:2, :].astype(jnp.int32))
        g_row = acc_row[2:3, :]
        k_col = (acc_col[:, 0:1].astype(jnp.int32) * 4096
                 + acc_col[:, 1:2].astype(jnp.int32))

        rio = lax.broadcasted_iota(jnp.int32, (K, K), 0)
        sio = lax.broadcasted_iota(jnp.int32, (K, K), 1)
        gtm = (k_row > k_col) | ((k_row == k_col) & (sio < rio))
        rank_col = jnp.sum(jnp.where(gtm, 1.0, 0.0), axis=1,
                           keepdims=True)
        O2 = jnp.where(rank_col == sio.astype(jnp.float32), 1.0, 0.0)
        idx_out = lax.dot_general(g_row, O2, (((1,), (0,)), ((), ())),
                                  precision=lax.Precision.HIGHEST,
                                  preferred_element_type=jnp.float32)
        rows.append(idx_out + float(b * T))
    idx_ref[...] = jnp.concatenate(rows, axis=0).astype(jnp.int32)


def _topk_global_idx(scores):
    B, T = scores.shape
    return pl.pallas_call(
        _topk_body,
        out_shape=jax.ShapeDtypeStruct((B, CAPACITY), jnp.int32),
    )(scores)


def _sc_mesh_info():
    info = plsc.get_sparse_core_info()
    return (plsc.VectorSubcoreMesh(core_axis_name="c", subcore_axis_name="s"),
            info.num_cores, info.num_cores * info.num_subcores)


def _make_sc_gather(V, D, Bn):
    mesh, nc, nw = _sc_mesh_info()
    b_per_w = Bn // nw

    @functools.partial(
        pl.kernel, mesh=mesh,
        out_type=jax.ShapeDtypeStruct((Bn, D), jnp.float32),
        scratch_types=[
            pltpu.VMEM((b_per_w,), jnp.int32),
            pltpu.VMEM((b_per_w, D), jnp.float32),
            pltpu.SemaphoreType.DMA,
        ],
    )
    def gather_k(table_hbm, idx_hbm, out_hbm, idx_v, rows_v, sem):
        wid = lax.axis_index("s") * nc + lax.axis_index("c")
        base = wid * b_per_w
        pltpu.sync_copy(idx_hbm.at[pl.ds(base, b_per_w)], idx_v)
        pltpu.async_copy(table_hbm.at[idx_v], rows_v, sem).wait()
        pltpu.sync_copy(rows_v, out_hbm.at[pl.ds(base, b_per_w)])

    return gather_k


def _make_sc_scatter(D, Bn):
    mesh, nc, nw = _sc_mesh_info()
    b_per_w = Bn // nw

    @functools.partial(
        pl.kernel, mesh=mesh,
        out_type=(),
        scratch_types=[
            pltpu.VMEM((b_per_w,), jnp.int32),
            pltpu.VMEM((b_per_w, D), jnp.float32),
            pltpu.SemaphoreType.DMA,
        ],
    )
    def scatter_k(rows_hbm, idx_hbm, out_ref, idx_v, rows_v, sem):
        wid = lax.axis_index("s") * nc + lax.axis_index("c")
        base = wid * b_per_w
        pltpu.sync_copy(idx_hbm.at[pl.ds(base, b_per_w)], idx_v)
        pltpu.sync_copy(rows_hbm.at[pl.ds(base, b_per_w)], rows_v)
        pltpu.async_copy(rows_v, out_ref.at[idx_v], sem).wait()

    return scatter_k


def _gather_rows(table, idx_flat):
    V, D = table.shape
    return _make_sc_gather(V, D, idx_flat.shape[0])(table, idx_flat)


def _scatter_rows(out_ref, rows, idx_flat):
    _make_sc_scatter(rows.shape[1], rows.shape[0])(rows, idx_flat, out_ref)


def _rms(h, g, eps=1e-6):
    norm = lax.rsqrt(jnp.mean(h * h, axis=-1, keepdims=True) + eps)
    return h * norm * g


def _qkv_body(x_ref, g1_ref, wqkv_ref, qkv_ref):
    h = _rms(x_ref[...], g1_ref[...])
    qkv_ref[...] = lax.dot_general(h, wqkv_ref[...],
                                   (((1,), (0,)), ((), ())),
                                   preferred_element_type=jnp.float32)


def _qkv_proj(xs, g1, Wqkv):
    N, D = xs.shape
    TM = 512
    return pl.pallas_call(
        _qkv_body,
        grid=(N // TM,),
        in_specs=[
            pl.BlockSpec((TM, D), lambda i: (i, 0)),
            pl.BlockSpec((1, D), lambda i: (0, 0)),
            pl.BlockSpec(Wqkv.shape, lambda i: (0, 0)),
        ],
        out_specs=pl.BlockSpec((TM, 3 * D), lambda i: (i, 0)),
        out_shape=jax.ShapeDtypeStruct((N, 3 * D), jnp.float32),
    )(xs, g1, Wqkv)


def _attn_body(q_ref, k_ref, v_ref, o_ref):
    Tn = q_ref.shape[0]
    scale = 1.0 / math.sqrt(64.0)
    rio = lax.broadcasted_iota(jnp.int32, (Tn, Tn), 0)
    cio = lax.broadcasted_iota(jnp.int32, (Tn, Tn), 1)
    outs = []
    for j in range(2):
        q = q_ref[:, j * 64:(j + 1) * 64]
        k = k_ref[:, j * 64:(j + 1) * 64]
        v = v_ref[:, j * 64:(j + 1) * 64]
        s = lax.dot_general(q, k, (((1,), (1,)), ((), ())),
                            preferred_element_type=jnp.float32) * scale
        s = jnp.where(cio > rio, -1e30, s)
        mx = jnp.max(s, axis=1, keepdims=True)
        e = jnp.exp(s - mx)
        p = e / jnp.sum(e, axis=1, keepdims=True)
        outs.append(lax.dot_general(p, v, (((1,), (0,)), ((), ())),
                                    preferred_element_type=jnp.float32))
    o_ref[...] = jnp.concatenate(outs, axis=1)


def _attention(qkv, B, D):
    N = qkv.shape[0]
    Tn = N // B
    return pl.pallas_call(
        _attn_body,
        grid=(B, N_HEADS // 2),
        in_specs=[
            pl.BlockSpec((Tn, 128), lambda b, h: (b, h)),
            pl.BlockSpec((Tn, 128), lambda b, h: (b, h + N_HEADS // 2)),
            pl.BlockSpec((Tn, 128), lambda b, h: (b, h + N_HEADS)),
        ],
        out_specs=pl.BlockSpec((Tn, 128), lambda b, h: (b, h)),
        out_shape=jax.ShapeDtypeStruct((N, D), jnp.float32),
    )(qkv, qkv, qkv)


def _ffn_body(x_ref, a_ref, g2_ref, wo_ref, w1_ref, w2_ref, w3_ref, o_ref):
    xs = x_ref[...]
    y = xs + lax.dot_general(a_ref[...], wo_ref[...],
                             (((1,), (0,)), ((), ())),
                             preferred_element_type=jnp.float32)
    h2 = _rms(y, g2_ref[...])
    a = lax.dot_general(h2, w1_ref[...], (((1,), (0,)), ((), ())),
                        preferred_element_type=jnp.float32)
    b = lax.dot_general(h2, w2_ref[...], (((1,), (0,)), ((), ())),
                        preferred_element_type=jnp.float32)
    act = (a / (1.0 + jnp.exp(-a))) * b
    ff = lax.dot_general(act, w3_ref[...], (((1,), (0,)), ((), ())),
                         preferred_element_type=jnp.float32)
    o_ref[...] = y + ff


def _out_ffn(xs, attn, g2, Wo, W1, W2, W3):
    N, D = xs.shape
    F = W1.shape[1]
    TM = 512
    return pl.pallas_call(
        _ffn_body,
        grid=(N // TM,),
        in_specs=[
            pl.BlockSpec((TM, D), lambda i: (i, 0)),
            pl.BlockSpec((TM, D), lambda i: (i, 0)),
            pl.BlockSpec((1, D), lambda i: (0, 0)),
            pl.BlockSpec((D, D), lambda i: (0, 0)),
            pl.BlockSpec((D, F), lambda i: (0, 0)),
            pl.BlockSpec((D, F), lambda i: (0, 0)),
            pl.BlockSpec((F, D), lambda i: (0, 0)),
        ],
        out_specs=pl.BlockSpec((TM, D), lambda i: (i, 0)),
        out_shape=jax.ShapeDtypeStruct((N, D), jnp.float32),
    )(xs, attn, g2, Wo, W1, W2, W3)


def kernel(x, Wr, g1, g2, Wqkv, Wo, W1, W2, W3):
    B, T, D = x.shape
    out, scores = _scores_and_copy(x, Wr)
    idx_g = _topk_global_idx(scores)
    idx_flat = idx_g.reshape(B * CAPACITY)
    xs = _gather_rows(x.reshape(B * T, D), idx_flat)
    qkv = _qkv_proj(xs, g1.reshape(1, D), Wqkv)
    attn = _attention(qkv, B, D)
    xproc = _out_ffn(xs, attn, g2.reshape(1, D), Wo, W1, W2, W3)
    out_ref = jax.new_ref(out.reshape(B * T, D))
    _scatter_rows(out_ref, xproc, idx_flat)
    return jax.freeze(out_ref).reshape(B, T, D)


# speedup vs baseline: 3.0556x; 2.0729x over previous
import functools
import math

import jax
import jax.numpy as jnp
from jax import lax
from jax.experimental import pallas as pl
from jax.experimental.pallas import tpu as pltpu
from jax.experimental.pallas import tpu_sc as plsc

N_HEADS = 12
CAPACITY = 1024
_INT_MIN_PY = -2147483648


def _scores_copy_body(x_ref, wr_ref, out_ref, s_ref):
    xb = x_ref[...]
    out_ref[...] = xb
    Bb, TB, D = xb.shape
    s = lax.dot_general(xb.reshape(Bb * TB, D), wr_ref[...],
                        (((1,), (0,)), ((), ())),
                        precision=lax.Precision.HIGHEST,
                        preferred_element_type=jnp.float32)
    s_ref[...] = s.reshape(Bb, TB)


def _scores_and_copy(x, Wr):
    B, T, D = x.shape
    TB = 512
    grid = (T // TB,)
    out, scores = pl.pallas_call(
        _scores_copy_body,
        grid=grid,
        in_specs=[
            pl.BlockSpec((B, TB, D), lambda i: (0, i, 0)),
            pl.BlockSpec((D, 1), lambda i: (0, 0)),
        ],
        out_specs=[
            pl.BlockSpec((B, TB, D), lambda i: (0, i, 0)),
            pl.BlockSpec((B, TB), lambda i: (0, i)),
        ],
        out_shape=[
            jax.ShapeDtypeStruct((B, T, D), jnp.float32),
            jax.ShapeDtypeStruct((B, T), jnp.float32),
        ],
    )(x, Wr)
    return out, scores


def _roll(x, d, axis):
    d = d % x.shape[axis]
    if d == 0:
        return x
    if axis == 0:
        return jnp.concatenate([x[d:, :], x[:d, :]], axis=0)
    return jnp.concatenate([x[:, d:], x[:, :d]], axis=1)


def _topk_body(s_ref, idx_ref):
    s = s_ref[...]
    B, T = s.shape
    K = CAPACITY
    INT_MIN = jnp.int32(_INT_MIN_PY)
    bits = lax.bitcast_convert_type(s, jnp.int32)
    key = jnp.where(bits >= 0, bits, INT_MIN - bits)

    R, C = T // 128, 128
    LOG = (T - 1).bit_length()
    rows_io = lax.broadcasted_iota(jnp.int32, (R, C), 0)
    cols_io = lax.broadcasted_iota(jnp.int32, (R, C), 1)
    lin = rows_io * C + cols_io
    up_masks = [(lin & (1 << j)) == 0 for j in range(LOG)]
    desc_masks = [((lin >> kk) & 1) == 0 for kk in range(1, LOG)]

    out_rows = []
    for b in range(B):
        k_arr = key[b].reshape(R, C)
        g_arr = lin
        for kk in range(1, LOG + 1):
            for j in range(kk - 1, -1, -1):
                d = 1 << j
                up = up_masks[j]
                if d < C:
                    pk = jnp.where(up, _roll(k_arr, d, 1), _roll(k_arr, -d, 1))
                    pg = jnp.where(up, _roll(g_arr, d, 1), _roll(g_arr, -d, 1))
                else:
                    m = d // C
                    pk = jnp.where(up, _roll(k_arr, m, 0), _roll(k_arr, -m, 0))
                    pg = jnp.where(up, _roll(g_arr, m, 0), _roll(g_arr, -m, 0))
                beats = (k_arr > pk) | ((k_arr == pk) & (g_arr < pg))
                if kk == LOG:
                    keep = beats == up
                else:
                    keep = beats == (up == desc_masks[kk - 1])
                k_arr = jnp.where(keep, k_arr, pk)
                g_arr = jnp.where(keep, g_arr, pg)
        top = g_arr[:K // C, :].reshape(1, K)
        out_rows.append(top + b * T)
    idx_ref[...] = jnp.concatenate(out_rows, axis=0)


def _topk_global_idx(scores):
    B, T = scores.shape
    return pl.pallas_call(
        _topk_body,
        out_shape=jax.ShapeDtypeStruct((B, CAPACITY), jnp.int32),
    )(scores)


def _sc_mesh_info():
    info = plsc.get_sparse_core_info()
    return (plsc.VectorSubcoreMesh(core_axis_name="c", subcore_axis_name="s"),
            info.num_cores, info.num_cores * info.num_subcores)


def _make_sc_gather(V, D, Bn):
    mesh, nc, nw = _sc_mesh_info()
    b_per_w = Bn // nw

    @functools.partial(
        pl.kernel, mesh=mesh,
        out_type=jax.ShapeDtypeStruct((Bn, D), jnp.float32),
        scratch_types=[
            pltpu.VMEM((b_per_w,), jnp.int32),
            pltpu.VMEM((b_per_w, D), jnp.float32),
            pltpu.SemaphoreType.DMA,
        ],
    )
    def gather_k(table_hbm, idx_hbm, out_hbm, idx_v, rows_v, sem):
        wid = lax.axis_index("s") * nc + lax.axis_index("c")
        base = wid * b_per_w
        pltpu.sync_copy(idx_hbm.at[pl.ds(base, b_per_w)], idx_v)
        pltpu.async_copy(table_hbm.at[idx_v], rows_v, sem).wait()
        pltpu.sync_copy(rows_v, out_hbm.at[pl.ds(base, b_per_w)])

    return gather_k


def _make_sc_scatter(D, Bn):
    mesh, nc, nw = _sc_mesh_info()
    b_per_w = Bn // nw

    @functools.partial(
        pl.kernel, mesh=mesh,
        out_type=(),
        scratch_types=[
            pltpu.VMEM((b_per_w,), jnp.int32),
            pltpu.VMEM((b_per_w, D), jnp.float32),
            pltpu.SemaphoreType.DMA,
        ],
    )
    def scatter_k(rows_hbm, idx_hbm, out_ref, idx_v, rows_v, sem):
        wid = lax.axis_index("s") * nc + lax.axis_index("c")
        base = wid * b_per_w
        pltpu.sync_copy(idx_hbm.at[pl.ds(base, b_per_w)], idx_v)
        pltpu.sync_copy(rows_hbm.at[pl.ds(base, b_per_w)], rows_v)
        pltpu.async_copy(rows_v, out_ref.at[idx_v], sem).wait()

    return scatter_k


def _gather_rows(table, idx_flat):
    V, D = table.shape
    return _make_sc_gather(V, D, idx_flat.shape[0])(table, idx_flat)


def _scatter_rows(out_ref, rows, idx_flat):
    _make_sc_scatter(rows.shape[1], rows.shape[0])(rows, idx_flat, out_ref)


def _rms(h, g, eps=1e-6):
    norm = lax.rsqrt(jnp.mean(h * h, axis=-1, keepdims=True) + eps)
    return h * norm * g


def _qkv_body(x_ref, g1_ref, wqkv_ref, qkv_ref):
    h = _rms(x_ref[...], g1_ref[...])
    qkv_ref[...] = lax.dot_general(h, wqkv_ref[...],
                                   (((1,), (0,)), ((), ())),
                                   preferred_element_type=jnp.float32)


def _qkv_proj(xs, g1, Wqkv):
    N, D = xs.shape
    TM = 512
    return pl.pallas_call(
        _qkv_body,
        grid=(N // TM,),
        in_specs=[
            pl.BlockSpec((TM, D), lambda i: (i, 0)),
            pl.BlockSpec((1, D), lambda i: (0, 0)),
            pl.BlockSpec(Wqkv.shape, lambda i: (0, 0)),
        ],
        out_specs=pl.BlockSpec((TM, 3 * D), lambda i: (i, 0)),
        out_shape=jax.ShapeDtypeStruct((N, 3 * D), jnp.float32),
    )(xs, g1, Wqkv)


def _attn_body(q_ref, k_ref, v_ref, o_ref):
    Tn = q_ref.shape[0]
    scale = 1.0 / math.sqrt(64.0)
    rio = lax.broadcasted_iota(jnp.int32, (Tn, Tn), 0)
    cio = lax.broadcasted_iota(jnp.int32, (Tn, Tn), 1)
    outs = []
    for j in range(2):
        q = q_ref[:, j * 64:(j + 1) * 64]
        k = k_ref[:, j * 64:(j + 1) * 64]
        v = v_ref[:, j * 64:(j + 1) * 64]
        s = lax.dot_general(q, k, (((1,), (1,)), ((), ())),
                            preferred_element_type=jnp.float32) * scale
        s = jnp.where(cio > rio, -1e30, s)
        mx = jnp.max(s, axis=1, keepdims=True)
        e = jnp.exp(s - mx)
        p = e / jnp.sum(e, axis=1, keepdims=True)
        outs.append(lax.dot_general(p, v, (((1,), (0,)), ((), ())),
                                    preferred_element_type=jnp.float32))
    o_ref[...] = jnp.concatenate(outs, axis=1)


def _attention(qkv, B, D):
    N = qkv.shape[0]
    Tn = N // B
    return pl.pallas_call(
        _attn_body,
        grid=(B, N_HEADS // 2),
        in_specs=[
            pl.BlockSpec((Tn, 128), lambda b, h: (b, h)),
            pl.BlockSpec((Tn, 128), lambda b, h: (b, h + N_HEADS // 2)),
            pl.BlockSpec((Tn, 128), lambda b, h: (b, h + N_HEADS)),
        ],
        out_specs=pl.BlockSpec((Tn, 128), lambda b, h: (b, h)),
        out_shape=jax.ShapeDtypeStruct((N, D), jnp.float32),
    )(qkv, qkv, qkv)


def _ffn_body(x_ref, a_ref, g2_ref, wo_ref, w1_ref, w2_ref, w3_ref, o_ref):
    xs = x_ref[...]
    y = xs + lax.dot_general(a_ref[...], wo_ref[...],
                             (((1,), (0,)), ((), ())),
                             preferred_element_type=jnp.float32)
    h2 = _rms(y, g2_ref[...])
    a = lax.dot_general(h2, w1_ref[...], (((1,), (0,)), ((), ())),
                        preferred_element_type=jnp.float32)
    b = lax.dot_general(h2, w2_ref[...], (((1,), (0,)), ((), ())),
                        preferred_element_type=jnp.float32)
    act = (a / (1.0 + jnp.exp(-a))) * b
    ff = lax.dot_general(act, w3_ref[...], (((1,), (0,)), ((), ())),
                         preferred_element_type=jnp.float32)
    o_ref[...] = y + ff


def _out_ffn(xs, attn, g2, Wo, W1, W2, W3):
    N, D = xs.shape
    F = W1.shape[1]
    TM = 512
    return pl.pallas_call(
        _ffn_body,
        grid=(N // TM,),
        in_specs=[
            pl.BlockSpec((TM, D), lambda i: (i, 0)),
            pl.BlockSpec((TM, D), lambda i: (i, 0)),
            pl.BlockSpec((1, D), lambda i: (0, 0)),
            pl.BlockSpec((D, D), lambda i: (0, 0)),
            pl.BlockSpec((D, F), lambda i: (0, 0)),
            pl.BlockSpec((D, F), lambda i: (0, 0)),
            pl.BlockSpec((F, D), lambda i: (0, 0)),
        ],
        out_specs=pl.BlockSpec((TM, D), lambda i: (i, 0)),
        out_shape=jax.ShapeDtypeStruct((N, D), jnp.float32),
    )(xs, attn, g2, Wo, W1, W2, W3)


def kernel(x, Wr, g1, g2, Wqkv, Wo, W1, W2, W3):
    B, T, D = x.shape
    out, scores = _scores_and_copy(x, Wr)
    idx_g = _topk_global_idx(scores)
    idx_flat = idx_g.reshape(B * CAPACITY)
    xs = _gather_rows(x.reshape(B * T, D), idx_flat)
    qkv = _qkv_proj(xs, g1.reshape(1, D), Wqkv)
    attn = _attention(qkv, B, D)
    xproc = _out_ffn(xs, attn, g2.reshape(1, D), Wo, W1, W2, W3)
    out_ref = jax.new_ref(out.reshape(B * T, D))
    _scatter_rows(out_ref, xproc, idx_flat)
    return jax.freeze(out_ref).reshape(B, T, D)
